# Initial kernel scaffold; baseline (speedup 1.0000x reference)
import functools

import jax
import jax.numpy as jnp
from jax import lax
from jax.experimental import pallas as pl
from jax.experimental.pallas import tpu as pltpu
from jax.experimental.pallas import tpu_sc as plsc

D = 64
ROLES = 7
NDOT = 6
NC = 2
NS = 16
L = 16
NW = NC * NS
B = 16384
BW = B // NW
CB = 64
NCHUNK = BW // CB
RPC = CB * ROLES
IDX_W = 112
NSUB = RPC // IDX_W

_mesh = plsc.VectorSubcoreMesh(
    core_axis_name="c", subcore_axis_name="s", num_cores=NC, num_subcores=NS)


@functools.partial(
    pl.kernel,
    out_type=jax.ShapeDtypeStruct((NW, NDOT, BW), jnp.float32),
    mesh=_mesh,
    scratch_types=[
        pltpu.VMEM((NSUB, IDX_W), jnp.int32),
        pltpu.VMEM((NSUB, IDX_W), jnp.int32),
        pltpu.VMEM((RPC, D), jnp.float32),
        pltpu.VMEM((RPC, D), jnp.float32),
        pltpu.VMEM((RPC, 2), jnp.float32),
        pltpu.VMEM((NDOT, CB), jnp.float32),
        pltpu.SemaphoreType.DMA,
    ],
)
def _sc_scores(idx0_hbm, idx1_hbm, t0_hbm, t1_hbm, ww_hbm, out_hbm,
               idx0_v, idx1_v, buf0, buf1, wbuf, obuf, sem):
    cid = lax.axis_index("c")
    sid = lax.axis_index("s")
    wid = sid * NC + cid
    zero_i = jnp.zeros((L,), jnp.int32)
    one_i = jnp.ones((L,), jnp.int32)
    lane = lax.iota(jnp.int32, L)
    for c in range(NCHUNK):
        sub0 = (wid * NCHUNK + c) * NSUB
        pltpu.sync_copy(idx0_hbm.at[pl.ds(sub0, NSUB), :], idx0_v)
        pltpu.sync_copy(idx1_hbm.at[pl.ds(sub0, NSUB), :], idx1_v)
        handles = []
        for j in range(NSUB):
            rows = pl.ds(j * IDX_W, IDX_W)
            handles.append(pltpu.async_copy(t0_hbm.at[idx0_v.at[j]], buf0.at[rows, :], sem))
            handles.append(pltpu.async_copy(t1_hbm.at[idx1_v.at[j]], buf1.at[rows, :], sem))
            handles.append(pltpu.async_copy(ww_hbm.at[idx0_v.at[j]], wbuf.at[rows, :], sem))
        for h in handles:
            h.wait()
        for g in range(CB // L):
            slots = [(lane + g * L) * ROLES + r for r in range(ROLES)]
            p0, p1 = [], []
            for r in range(ROLES):
                w0 = plsc.load_gather(wbuf, [slots[r], zero_i])
                w1 = plsc.load_gather(wbuf, [slots[r], one_i])
                a = 1.0 / (1.0 + jnp.exp(w1 - w0))
                p0.append(a)
                p1.append(1.0 - a)

            def body(d, accs, slots=slots, p0=p0, p1=p1):
                dv = zero_i + d
                e = []
                for r in range(ROLES):
                    g0 = plsc.load_gather(buf0, [slots[r], dv])
                    g1 = plsc.load_gather(buf1, [slots[r], dv])
                    e.append(p0[r] * g0 + p1[r] * g1)
                out = [accs[0] + e[0] * e[1]]
                for i in range(NDOT - 1):
                    out.append(accs[1 + i] + e[0] * e[2 + i])
                return tuple(out)

            acc = lax.fori_loop(0, D, body, (jnp.zeros((L,), jnp.float32),) * NDOT)
            for k in range(NDOT):
                obuf[k, pl.ds(g * L, L)] = acc[k]
        pltpu.sync_copy(obuf, out_hbm.at[wid, :, pl.ds(c * CB, CB)])


def _loss_body(s_ref, o_ref):
    x = jnp.clip(s_ref[...], -10.0, 10.0)
    seg = (lax.broadcasted_iota(jnp.int32, x.shape, 0) * 2
           + lax.broadcasted_iota(jnp.int32, x.shape, 1) // BW)
    z = jnp.where(seg % NDOT == 0, -x, x)
    o_ref[0, 0] = jnp.sum(jnp.logaddexp(z, 0.0)) * (1.0 / B)


_loss = pl.pallas_call(
    _loss_body,
    out_shape=jax.ShapeDtypeStruct((1, 1), jnp.float32),
    out_specs=pl.BlockSpec(memory_space=pltpu.SMEM),
)


def kernel(centers, contexts, neg_contexts, W_center0, W_center1, W_weights):
    centers = centers.astype(jnp.int32)
    contexts = contexts.astype(jnp.int32)
    neg_contexts = neg_contexts.astype(jnp.int32)
    idx0 = jnp.concatenate(
        [centers[:, 0:1], contexts[:, 0:1], neg_contexts[:, 0::2]], axis=1)
    idx1 = jnp.concatenate(
        [centers[:, 1:2], contexts[:, 1:2], neg_contexts[:, 1::2]], axis=1)
    idx0 = idx0.reshape(B * ROLES // IDX_W, IDX_W)
    idx1 = idx1.reshape(B * ROLES // IDX_W, IDX_W)
    scores = _sc_scores(idx0, idx1, W_center0, W_center1, W_weights)
    return _loss(scores.reshape(NW * NDOT // 2, BW * 2))[0, 0]


# speedup vs baseline: 1.4449x; 1.4449x over previous
import functools

import jax
import jax.numpy as jnp
from jax import lax
from jax.experimental import pallas as pl
from jax.experimental.pallas import tpu as pltpu
from jax.experimental.pallas import tpu_sc as plsc

D = 64
ROLES = 7
NDOT = 6
NC = 2
NS = 16
L = 16
NW = NC * NS
B = 16384
BW = B // NW
CB = 64
NCHUNK = BW // CB
RPC = CB * ROLES
IDX_W = 112
NSUB = RPC // IDX_W

_mesh = plsc.VectorSubcoreMesh(
    core_axis_name="c", subcore_axis_name="s", num_cores=NC, num_subcores=NS)


@functools.partial(
    pl.kernel,
    out_type=jax.ShapeDtypeStruct((NW, NDOT, BW), jnp.float32),
    mesh=_mesh,
    compiler_params=pltpu.CompilerParams(
        needs_layout_passes=False, use_tc_tiling_on_sc=False),
    scratch_types=[
        pltpu.VMEM((NSUB, IDX_W), jnp.int32),
        pltpu.VMEM((NSUB, IDX_W), jnp.int32),
        pltpu.VMEM((RPC, D), jnp.float32),
        pltpu.VMEM((RPC, D), jnp.float32),
        pltpu.VMEM((RPC, 2), jnp.float32),
        pltpu.VMEM((NDOT, CB), jnp.float32),
        pltpu.SemaphoreType.DMA,
    ],
)
def _sc_scores(idx0_hbm, idx1_hbm, t0_hbm, t1_hbm, ww_hbm, out_hbm,
               idx0_v, idx1_v, buf0, buf1, wbuf, obuf, sem):
    cid = lax.axis_index("c")
    sid = lax.axis_index("s")
    wid = sid * NC + cid
    zero_i = jnp.zeros((L,), jnp.int32)
    one_i = jnp.ones((L,), jnp.int32)
    lane = lax.iota(jnp.int32, L)
    for c in range(NCHUNK):
        sub0 = (wid * NCHUNK + c) * NSUB
        pltpu.sync_copy(idx0_hbm.at[pl.ds(sub0, NSUB), :], idx0_v)
        pltpu.sync_copy(idx1_hbm.at[pl.ds(sub0, NSUB), :], idx1_v)
        handles = []
        for j in range(NSUB):
            rows = pl.ds(j * IDX_W, IDX_W)
            handles.append(pltpu.async_copy(t0_hbm.at[idx0_v.at[j]], buf0.at[rows, :], sem))
            handles.append(pltpu.async_copy(t1_hbm.at[idx1_v.at[j]], buf1.at[rows, :], sem))
            handles.append(pltpu.async_copy(ww_hbm.at[idx0_v.at[j]], wbuf.at[rows, :], sem))
        for h in handles:
            h.wait()
        for g in range(CB // L):
            slots = [(lane + g * L) * ROLES + r for r in range(ROLES)]
            p0, p1 = [], []
            for r in range(ROLES):
                w0 = plsc.load_gather(wbuf, [slots[r], zero_i])
                w1 = plsc.load_gather(wbuf, [slots[r], one_i])
                a = 1.0 / (1.0 + jnp.exp(w1 - w0))
                p0.append(a)
                p1.append(1.0 - a)

            def body(d, accs, slots=slots, p0=p0, p1=p1):
                dv = zero_i + d
                e = []
                for r in range(ROLES):
                    g0 = plsc.load_gather(buf0, [slots[r], dv])
                    g1 = plsc.load_gather(buf1, [slots[r], dv])
                    e.append(p0[r] * g0 + p1[r] * g1)
                out = [accs[0] + e[0] * e[1]]
                for i in range(NDOT - 1):
                    out.append(accs[1 + i] + e[0] * e[2 + i])
                return tuple(out)

            acc = lax.fori_loop(0, D, body, (jnp.zeros((L,), jnp.float32),) * NDOT)
            for k in range(NDOT):
                obuf[k, pl.ds(g * L, L)] = acc[k]
        pltpu.sync_copy(obuf, out_hbm.at[wid, :, pl.ds(c * CB, CB)])


def _loss_body(s_ref, o_ref):
    x = jnp.clip(s_ref[...], -10.0, 10.0)
    seg = (lax.broadcasted_iota(jnp.int32, x.shape, 0) * 2
           + lax.broadcasted_iota(jnp.int32, x.shape, 1) // BW)
    z = jnp.where(seg % NDOT == 0, -x, x)
    o_ref[0, 0] = jnp.sum(jnp.logaddexp(z, 0.0)) * (1.0 / B)


_loss = pl.pallas_call(
    _loss_body,
    out_shape=jax.ShapeDtypeStruct((1, 1), jnp.float32),
    out_specs=pl.BlockSpec(memory_space=pltpu.SMEM),
)


def kernel(centers, contexts, neg_contexts, W_center0, W_center1, W_weights):
    centers = centers.astype(jnp.int32)
    contexts = contexts.astype(jnp.int32)
    neg_contexts = neg_contexts.astype(jnp.int32)
    idx0 = jnp.concatenate(
        [centers[:, 0:1], contexts[:, 0:1], neg_contexts[:, 0::2]], axis=1)
    idx1 = jnp.concatenate(
        [centers[:, 1:2], contexts[:, 1:2], neg_contexts[:, 1::2]], axis=1)
    idx0 = idx0.reshape(B * ROLES // IDX_W, IDX_W)
    idx1 = idx1.reshape(B * ROLES // IDX_W, IDX_W)
    scores = _sc_scores(idx0, idx1, W_center0, W_center1, W_weights)
    return _loss(scores.reshape(NW * NDOT // 2, BW * 2))[0, 0]


# speedup vs baseline: 2.4473x; 1.6937x over previous
import functools

import jax
import jax.numpy as jnp
from jax import lax
from jax.experimental import pallas as pl
from jax.experimental.pallas import tpu as pltpu
from jax.experimental.pallas import tpu_sc as plsc

D = 64
NFIELD = 2
NNEG = 5
ROLES = 7
NDOT = 6
NC = 2
NS = 16
L = 16
NW = NC * NS
B = 16384
BW = B // NW
CB = 64
NCHUNK = BW // CB
RPC = CB * ROLES
IDX_W = 112
NSUB = RPC // IDX_W

_mesh = plsc.VectorSubcoreMesh(
    core_axis_name="c", subcore_axis_name="s", num_cores=NC, num_subcores=NS)


@functools.partial(
    pl.kernel,
    out_type=jax.ShapeDtypeStruct((NW, NDOT, BW), jnp.float32),
    mesh=_mesh,
    compiler_params=pltpu.CompilerParams(
        needs_layout_passes=False, use_tc_tiling_on_sc=False),
    scratch_types=[
        pltpu.VMEM((BW // 4, NFIELD), jnp.int32),
        pltpu.VMEM((BW // 4, NFIELD), jnp.int32),
        pltpu.VMEM((BW // 4, NNEG * NFIELD), jnp.int32),
        pltpu.VMEM((NSUB, IDX_W), jnp.int32),
        pltpu.VMEM((NSUB, IDX_W), jnp.int32),
        pltpu.VMEM((2, RPC, D), jnp.float32),
        pltpu.VMEM((2, RPC, D), jnp.float32),
        pltpu.VMEM((2, RPC, NFIELD), jnp.float32),
        pltpu.VMEM((NDOT, CB), jnp.float32),
        pltpu.SemaphoreType.DMA,
        pltpu.SemaphoreType.DMA,
    ],
)
def _sc_scores(c_hbm, x_hbm, n_hbm, t0_hbm, t1_hbm, ww_hbm, out_hbm,
               cstage, xstage, nstage, idx0_v, idx1_v, buf0, buf1, wbuf,
               obuf, sem0, sem1):
    cid = lax.axis_index("c")
    sid = lax.axis_index("s")
    wid = sid * NC + cid
    zero_i = jnp.zeros((L,), jnp.int32)
    one_i = jnp.ones((L,), jnp.int32)
    lane = lax.iota(jnp.int32, L)
    sems = (sem0, sem1)
    half_chunks = NCHUNK // 4

    def build_and_issue(c, sl):
        if c % half_chunks == 0:
            base = wid * BW + (c // half_chunks) * (BW // 4)
            pltpu.sync_copy(c_hbm.at[pl.ds(base, BW // 4), :], cstage)
            pltpu.sync_copy(x_hbm.at[pl.ds(base, BW // 4), :], xstage)
            pltpu.sync_copy(n_hbm.at[pl.ds(base, BW // 4), :], nstage)
        for g in range(CB // L):
            ev = (c % half_chunks) * CB + g * L + lane
            for f, idx_v in ((0, idx0_v), (1, idx1_v)):
                fld = zero_i + f if f == 0 else one_i
                vals = [plsc.load_gather(cstage, [ev, fld]),
                        plsc.load_gather(xstage, [ev, fld])]
                for i in range(NNEG):
                    vals.append(plsc.load_gather(nstage, [ev, fld + 2 * i]))
                for r in range(ROLES):
                    p = r * CB + g * L
                    idx_v[p // IDX_W, pl.ds(p % IDX_W, L)] = vals[r]
        handles = []
        for j in range(NSUB):
            rows = pl.ds(j * IDX_W, IDX_W)
            handles.append(pltpu.async_copy(t0_hbm.at[idx0_v.at[j]], buf0.at[sl, rows, :], sems[sl]))
            handles.append(pltpu.async_copy(t1_hbm.at[idx1_v.at[j]], buf1.at[sl, rows, :], sems[sl]))
            handles.append(pltpu.async_copy(ww_hbm.at[idx0_v.at[j]], wbuf.at[sl, rows, :], sems[sl]))
        return handles

    pending = build_and_issue(0, 0)
    for c in range(NCHUNK):
        sl = c & 1
        for h in pending:
            h.wait()
        if c + 1 < NCHUNK:
            pending = build_and_issue(c + 1, 1 - sl)
        cbuf0, cbuf1, cwbuf = buf0.at[sl], buf1.at[sl], wbuf.at[sl]
        for g in range(CB // L):
            slots = [r * CB + g * L + lane for r in range(ROLES)]
            p0, p1 = [], []
            for r in range(ROLES):
                w0 = plsc.load_gather(cwbuf, [slots[r], zero_i])
                w1 = plsc.load_gather(cwbuf, [slots[r], one_i])
                a = 1.0 / (1.0 + jnp.exp(w1 - w0))
                p0.append(a)
                p1.append(1.0 - a)

            def body(d, accs, slots=slots, p0=p0, p1=p1, cbuf0=cbuf0, cbuf1=cbuf1):
                dv = (d & ~15) + (((d & 15) + lane) & 15)
                e = []
                for r in range(ROLES):
                    g0 = plsc.load_gather(cbuf0, [slots[r], dv])
                    g1 = plsc.load_gather(cbuf1, [slots[r], dv])
                    e.append(p0[r] * g0 + p1[r] * g1)
                out = [accs[0] + e[0] * e[1]]
                for i in range(NDOT - 1):
                    out.append(accs[1 + i] + e[0] * e[2 + i])
                return tuple(out)

            acc = lax.fori_loop(0, D, body, (jnp.zeros((L,), jnp.float32),) * NDOT)
            for k in range(NDOT):
                obuf[k, pl.ds(g * L, L)] = acc[k]
        pltpu.sync_copy(obuf, out_hbm.at[wid, :, pl.ds(c * CB, CB)])


def _loss_body(s_ref, o_ref):
    x = jnp.clip(s_ref[...], -10.0, 10.0)
    seg = (lax.broadcasted_iota(jnp.int32, x.shape, 0) * 2
           + lax.broadcasted_iota(jnp.int32, x.shape, 1) // BW)
    z = jnp.where(seg % NDOT == 0, -x, x)
    o_ref[0, 0] = jnp.sum(jnp.logaddexp(z, 0.0)) * (1.0 / B)


_loss = pl.pallas_call(
    _loss_body,
    out_shape=jax.ShapeDtypeStruct((1, 1), jnp.float32),
    out_specs=pl.BlockSpec(memory_space=pltpu.SMEM),
)


def kernel(centers, contexts, neg_contexts, W_center0, W_center1, W_weights):
    centers = centers.astype(jnp.int32)
    contexts = contexts.astype(jnp.int32)
    neg_contexts = neg_contexts.astype(jnp.int32)
    scores = _sc_scores(centers, contexts, neg_contexts,
                        W_center0, W_center1, W_weights)
    return _loss(scores.reshape(NW * NDOT // 2, BW * 2))[0, 0]


# speedup vs baseline: 3.3104x; 1.3527x over previous
import functools

import jax
import jax.numpy as jnp
from jax import lax
from jax.experimental import pallas as pl
from jax.experimental.pallas import tpu as pltpu
from jax.experimental.pallas import tpu_sc as plsc

D = 64
DP = 128
NFIELD = 2
ROLES = 7
NDOT = 6
NC = 2
NS = 16
L = 16
NW = NC * NS
B = 16384
VOCAB = 100000
BW = B // NW
CB = 32
NCHUNK = BW // CB
RPC = CB * ROLES
IDX_W = 112
NSUB = RPC // IDX_W

_mesh = plsc.VectorSubcoreMesh(
    core_axis_name="c", subcore_axis_name="s", num_cores=NC, num_subcores=NS)


@functools.partial(
    pl.kernel,
    out_type=jax.ShapeDtypeStruct((NW, NDOT, BW), jnp.float32),
    mesh=_mesh,
    compiler_params=pltpu.CompilerParams(
        needs_layout_passes=False, use_tc_tiling_on_sc=True),
    scratch_types=[
        pltpu.VMEM((NCHUNK * NSUB, IDX_W), jnp.int32),
        pltpu.VMEM((NCHUNK * NSUB, IDX_W), jnp.int32),
        pltpu.VMEM((2, RPC, DP), jnp.float32),
        pltpu.VMEM((2, RPC, DP), jnp.float32),
        pltpu.VMEM((NDOT, 4 * CB), jnp.float32),
        pltpu.SemaphoreType.DMA,
        pltpu.SemaphoreType.DMA,
    ],
)
def _sc_scores(idx0_hbm, idx1_hbm, t0_hbm, t1_hbm, out_hbm,
               idx0_v, idx1_v, buf0, buf1, obuf, sem0, sem1):
    cid = lax.axis_index("c")
    sid = lax.axis_index("s")
    wid = sid * NC + cid
    zero_i = jnp.zeros((L,), jnp.int32)
    one_i = jnp.ones((L,), jnp.int32)
    lane = lax.iota(jnp.int32, L)
    pltpu.sync_copy(idx0_hbm.at[pl.ds(wid * NCHUNK * NSUB, NCHUNK * NSUB), :], idx0_v)
    pltpu.sync_copy(idx1_hbm.at[pl.ds(wid * NCHUNK * NSUB, NCHUNK * NSUB), :], idx1_v)
    sems = (sem0, sem1)

    def issue(c, sl):
        handles = []
        for j in range(NSUB):
            rows = pl.ds(j * IDX_W, IDX_W)
            r = c * NSUB + j
            handles.append(pltpu.async_copy(
                t0_hbm.at[idx0_v.at[r]], buf0.at[sl, rows, :], sems[sl]))
            handles.append(pltpu.async_copy(
                t1_hbm.at[idx1_v.at[r]], buf1.at[sl, rows, :], sems[sl]))
        return handles

    pending = issue(0, 0)
    for c in range(NCHUNK):
        sl = c & 1
        for h in pending:
            h.wait()
        if c + 1 < NCHUNK:
            pending = issue(c + 1, 1 - sl)
        cbuf0, cbuf1 = buf0.at[sl], buf1.at[sl]
        for g in range(CB // L):
            slots = [(lane + g * L) * ROLES + r for r in range(ROLES)]
            p0, p1 = [], []
            for r in range(ROLES):
                w0 = plsc.load_gather(cbuf0, [slots[r], zero_i + D])
                w1 = plsc.load_gather(cbuf0, [slots[r], zero_i + (D + 1)])
                a = 1.0 / (1.0 + jnp.exp(w1 - w0))
                p0.append(a)
                p1.append(1.0 - a)

            def body(d, accs, slots=slots, p0=p0, p1=p1, cbuf0=cbuf0, cbuf1=cbuf1):
                dv = (d & ~15) + (((d & 15) + lane) & 15)
                e = []
                for r in range(ROLES):
                    g0 = plsc.load_gather(cbuf0, [slots[r], dv])
                    g1 = plsc.load_gather(cbuf1, [slots[r], dv])
                    e.append(p0[r] * g0 + p1[r] * g1)
                out = [accs[0] + e[0] * e[1]]
                for i in range(NDOT - 1):
                    out.append(accs[1 + i] + e[0] * e[2 + i])
                return tuple(out)

            acc = lax.fori_loop(0, D, body, (jnp.zeros((L,), jnp.float32),) * NDOT)
            for k in range(NDOT):
                obuf[k, pl.ds((c % 4) * CB + g * L, L)] = acc[k]
        if c % 4 == 3:
            pltpu.sync_copy(obuf, out_hbm.at[wid, :, pl.ds((c // 4) * 4 * CB, 4 * CB)])


def _loss_body(s_ref, o_ref):
    x = jnp.clip(s_ref[...], -10.0, 10.0)
    seg = (lax.broadcasted_iota(jnp.int32, x.shape, 0) * 2
           + lax.broadcasted_iota(jnp.int32, x.shape, 1) // BW)
    z = jnp.where(seg % NDOT == 0, -x, x)
    o_ref[0, 0] = jnp.sum(jnp.logaddexp(z, 0.0)) * (1.0 / B)


_loss = pl.pallas_call(
    _loss_body,
    out_shape=jax.ShapeDtypeStruct((1, 1), jnp.float32),
    out_specs=pl.BlockSpec(memory_space=pltpu.SMEM),
)


def kernel(centers, contexts, neg_contexts, W_center0, W_center1, W_weights):
    centers = centers.astype(jnp.int32)
    contexts = contexts.astype(jnp.int32)
    neg_contexts = neg_contexts.astype(jnp.int32)
    idx0 = jnp.concatenate(
        [centers[:, 0:1], contexts[:, 0:1], neg_contexts[:, 0::2]], axis=1)
    idx1 = jnp.concatenate(
        [centers[:, 1:2], contexts[:, 1:2], neg_contexts[:, 1::2]], axis=1)
    idx0 = idx0.reshape(B * ROLES // IDX_W, IDX_W)
    idx1 = idx1.reshape(B * ROLES // IDX_W, IDX_W)
    t0p = jnp.concatenate(
        [W_center0, W_weights,
         jnp.zeros((VOCAB, DP - D - NFIELD), jnp.float32)], axis=1)
    t1p = jnp.concatenate(
        [W_center1, jnp.zeros((VOCAB, DP - D), jnp.float32)], axis=1)
    scores = _sc_scores(idx0, idx1, t0p, t1p)
    return _loss(scores.reshape(NW * NDOT // 2, BW * 2))[0, 0]
